# baseline (speedup 1.0000x reference)
import jax
import jax.numpy as jnp
from jax import lax
from jax.experimental import pallas as pl
from jax.experimental.pallas import tpu as pltpu
from jax.experimental.pallas import tpu_sc as plsc

N_NODES = 10000
D_FEAT = 128
N_EDGES = 320000

_NC = 2
_NS = 16
_NW = _NC * _NS
_EPT = N_EDGES // _NW
_K = 80
_NPASS = 5
_EPP = _EPT // _NPASS
_CPP = _EPP // _K
_NBUF = 4
_N_PAD = 10240
_RPT = _N_PAD // _NS


def _scale_chunk(rows16, rowsf, vals_p, ci):
    @plsc.parallel_loop(0, _K, unroll=4)
    def kloop(k):
        vs = plsc.load_gather(vals_p, [jnp.full((16,), ci * _K + k, jnp.int32)])
        for d in range(D_FEAT // 32):
            packed = rows16[k, pl.ds(d * 16, 16)]
            pair = plsc.bitcast(packed, jnp.bfloat16)
            lo, hi = plsc.unpack(pair, format=plsc.PackFormat.INTERLEAVED)
            rowsf[k, pl.ds(d * 32, 16)] = lo * vs
            rowsf[k, pl.ds(d * 32 + 16, 16)] = hi * vs


def _sc_body(x_hbm, src_hbm, tgt_hbm, vals_hbm, out_hbm,
             acc_sh, b0, b1, b2, b3, f0, f1, src_p, tgt_p, vals_p,
             g0, g1, g2, g3, s0, s1):
    bufs = (b0, b1, b2, b3)
    fbufs = (f0, f1)
    gsems = (g0, g1, g2, g3)
    ssems = (s0, s1)
    c = lax.axis_index("c")
    s = lax.axis_index("s")
    wid = s * _NC + c

    zeros16 = jnp.zeros((16,), jnp.float32)

    def zbody(i, _):
        for d in range(D_FEAT // 16):
            f0[i, pl.ds(d * 16, 16)] = zeros16
        return 0

    lax.fori_loop(0, _K, zbody, 0)
    r0 = s * _RPT
    for j in range(_RPT // _K):
        pltpu.sync_copy(f0, acc_sh.at[pl.ds(r0 + j * _K, _K)])
    plsc.subcore_barrier()

    def gather_start(ci, b):
        pltpu.async_copy(x_hbm.at[src_p.at[pl.ds(ci * _K, _K)]],
                         bufs[b], gsems[b])

    def gather_wait(ci, b):
        pltpu.make_async_copy(x_hbm.at[src_p.at[pl.ds(ci * _K, _K)]],
                              bufs[b], gsems[b]).wait()

    def scat_fire(ci, f):
        for g in range(_K // 16):
            tv = tgt_p[pl.ds(ci * _K + g * 16, 16)]
            pltpu.async_copy(fbufs[f].at[pl.ds(g * 16, 16)], acc_sh.at[tv],
                             ssems[f], add=True)

    def scat_drain(ci, f):
        for g in range(_K // 16):
            tv = tgt_p[pl.ds(ci * _K + g * 16, 16)]
            pltpu.make_async_copy(fbufs[f].at[pl.ds(g * 16, 16)],
                                  acc_sh.at[tv], ssems[f]).wait()

    for ps in range(_NPASS):
        e0 = wid * _EPT + ps * _EPP
        pltpu.sync_copy(src_hbm.at[pl.ds(e0, _EPP)], src_p)
        pltpu.sync_copy(tgt_hbm.at[pl.ds(e0, _EPP)], tgt_p)
        pltpu.sync_copy(vals_hbm.at[pl.ds(e0, _EPP)], vals_p)

        for b in range(_NBUF - 1):
            gather_start(b, b)

        def step(q, b, guard_drain, guard_gather):
            f = b % 2
            gather_wait(q, b)
            if guard_gather:
                @pl.when(q + _NBUF - 1 < _CPP)
                def _():
                    gather_start(q + _NBUF - 1, (b + _NBUF - 1) % _NBUF)
            pf = (f + 1) % 2
            if guard_drain:
                @pl.when(q > 0)
                def _():
                    scat_drain(q - 1, pf)
            else:
                scat_drain(q - 1, pf)
            _scale_chunk(bufs[b], fbufs[f], vals_p, q)
            scat_fire(q, f)

        def pbody(p, _):
            for b in range(_NBUF):
                q = _NBUF * p + b
                step(q, b, guard_drain=(b == 0), guard_gather=True)
            return 0

        ntail = _CPP % _NBUF
        nfull = _CPP // _NBUF
        lax.fori_loop(0, nfull, pbody, 0)
        for t in range(ntail):
            q = nfull * _NBUF + t
            step(q, q % _NBUF, guard_drain=False, guard_gather=True)
        scat_drain(_CPP - 1, (_CPP - 1) % 2)

    plsc.subcore_barrier()

    for j in range(_RPT // _K):
        rr = r0 + j * _K
        pltpu.async_copy(acc_sh.at[pl.ds(rr, _K)],
                         out_hbm.at[c].at[pl.ds(rr, _K)], gsems[j % _NBUF])
    for j in range(_RPT // _K):
        rr = r0 + j * _K
        pltpu.make_async_copy(acc_sh.at[pl.ds(rr, _K)],
                              out_hbm.at[c].at[pl.ds(rr, _K)],
                              gsems[j % _NBUF]).wait()


def _tc_add_body(a_ref, b_ref, o_ref):
    o_ref[...] = a_ref[...] + b_ref[...]


def kernel(x_source, neighborhood_indices, neighborhood_values):
    tgt = neighborhood_indices[0]
    src = neighborhood_indices[1]
    x16 = jax.lax.bitcast_convert_type(
        x_source.astype(jnp.bfloat16)
        .reshape(N_NODES, D_FEAT // 32, 2, 16)
        .transpose(0, 1, 3, 2)
        .reshape(N_NODES, D_FEAT // 2, 2),
        jnp.int32)

    mesh = plsc.VectorSubcoreMesh(core_axis_name="c", subcore_axis_name="s")
    partials = pl.kernel(
        _sc_body,
        mesh=mesh,
        compiler_params=pltpu.CompilerParams(needs_layout_passes=False,
                                             use_tc_tiling_on_sc=False),
        out_type=jax.ShapeDtypeStruct((_NC, _N_PAD, D_FEAT), jnp.float32),
        scratch_types=[
            pltpu.VMEM_SHARED((_N_PAD, D_FEAT), jnp.float32),
            pltpu.VMEM((_K, D_FEAT // 2), jnp.int32),
            pltpu.VMEM((_K, D_FEAT // 2), jnp.int32),
            pltpu.VMEM((_K, D_FEAT // 2), jnp.int32),
            pltpu.VMEM((_K, D_FEAT // 2), jnp.int32),
            pltpu.VMEM((_K, D_FEAT), jnp.float32),
            pltpu.VMEM((_K, D_FEAT), jnp.float32),
            pltpu.VMEM((_EPP,), jnp.int32),
            pltpu.VMEM((_EPP,), jnp.int32),
            pltpu.VMEM((_EPP,), jnp.float32),
            pltpu.SemaphoreType.DMA,
            pltpu.SemaphoreType.DMA,
            pltpu.SemaphoreType.DMA,
            pltpu.SemaphoreType.DMA,
            pltpu.SemaphoreType.DMA,
            pltpu.SemaphoreType.DMA,
        ],
    )(x16, src, tgt, neighborhood_values)

    blk = 1000
    out = pl.pallas_call(
        _tc_add_body,
        out_shape=jax.ShapeDtypeStruct((N_NODES, D_FEAT), jnp.float32),
        grid=(N_NODES // blk,),
        in_specs=[
            pl.BlockSpec((blk, D_FEAT), lambda i: (i, 0)),
            pl.BlockSpec((blk, D_FEAT), lambda i: (i, 0)),
        ],
        out_specs=pl.BlockSpec((blk, D_FEAT), lambda i: (i, 0)),
    )(partials[0], partials[1])
    return out
